# baseline (speedup 1.0000x reference)
import functools
import jax
import jax.numpy as jnp
from jax import lax
from jax.experimental import pallas as pl
from jax.experimental.pallas import tpu as pltpu
from jax.experimental.pallas import tpu_sc as plsc

N_NODES_C = 10000
N_EDGES_C = 160000
D_C = 256

NC, NS = 2, 16
NW = NC * NS
NQ = 16
NG = 2
CG = D_C // NG
QN = N_NODES_C // NQ
EPW = N_EDGES_C // NW
GC = 200
EC = 320
NEG_INF = float("-inf")

_sc_mesh = plsc.VectorSubcoreMesh(
    core_axis_name="c", subcore_axis_name="s", num_cores=NC, num_subcores=NS
)


def _node_mm_body(x_ref, w1_ref, b1_ref, a_ref, b_ref):
    xb = x_ref[...]
    wb = w1_ref[D_C:, :]
    wa = w1_ref[:D_C, :] - wb
    a_ref[...] = (
        jnp.dot(xb, wa, preferred_element_type=jnp.float32) + b1_ref[...]
    )
    b_ref[...] = jnp.dot(xb, wb, preferred_element_type=jnp.float32)


def _node_mm(x, W1, b1):
    nb = 400
    grid = (N_NODES_C // nb,)
    return pl.pallas_call(
        _node_mm_body,
        grid=grid,
        in_specs=[
            pl.BlockSpec((nb, D_C), lambda i: (i, 0)),
            pl.BlockSpec((2 * D_C, D_C), lambda i: (0, 0)),
            pl.BlockSpec((1, D_C), lambda i: (0, 0)),
        ],
        out_specs=[
            pl.BlockSpec((nb, D_C), lambda i: (i, 0)),
            pl.BlockSpec((nb, D_C), lambda i: (i, 0)),
        ],
        out_shape=[
            jax.ShapeDtypeStruct((N_NODES_C, D_C), jnp.float32),
            jax.ShapeDtypeStruct((N_NODES_C, D_C), jnp.float32),
        ],
    )(x, W1, b1.reshape(1, D_C))


def _gather_body(a_hbm, b_hbm, dst_hbm, src_hbm, pa_hbm, pb_hbm,
                 didx, sidx, bufa, bufb, sema, semb):
    wid = lax.axis_index("s") * NC + lax.axis_index("c")
    e0 = wid * EPW

    def chunk(j, _):
        base = e0 + j * GC
        pltpu.sync_copy(dst_hbm.at[pl.ds(base, GC)], didx)
        pltpu.sync_copy(src_hbm.at[pl.ds(base, GC)], sidx)
        cpa = pltpu.async_copy(a_hbm.at[didx], bufa, sema)
        cpb = pltpu.async_copy(b_hbm.at[sidx], bufb, semb)
        cpa.wait()
        pltpu.sync_copy(bufa, pa_hbm.at[pl.ds(base, GC)])
        cpb.wait()
        pltpu.sync_copy(bufb, pb_hbm.at[pl.ds(base, GC)])
        return 0

    lax.fori_loop(0, EPW // GC, chunk, 0)


@functools.partial(
    pl.kernel,
    out_type=[
        jax.ShapeDtypeStruct((N_EDGES_C, D_C), jnp.float32),
        jax.ShapeDtypeStruct((N_EDGES_C, D_C), jnp.float32),
    ],
    mesh=_sc_mesh,
    scratch_types=[
        pltpu.VMEM((GC,), jnp.int32),
        pltpu.VMEM((GC,), jnp.int32),
        pltpu.VMEM((GC, D_C), jnp.float32),
        pltpu.VMEM((GC, D_C), jnp.float32),
        pltpu.SemaphoreType.DMA,
        pltpu.SemaphoreType.DMA,
    ],
)
def _edge_gather(a_hbm, b_hbm, dst_hbm, src_hbm, pa_hbm, pb_hbm,
                 didx, sidx, bufa, bufb, sema, semb):
    _gather_body(a_hbm, b_hbm, dst_hbm, src_hbm, pa_hbm, pb_hbm,
                 didx, sidx, bufa, bufb, sema, semb)


def _edge_mm_body(pa_ref, pb_ref, w2_ref, h_ref):
    m = jnp.maximum(pa_ref[...] + pb_ref[...], 0.0)
    h = jnp.dot(m, w2_ref[...], preferred_element_type=jnp.float32)
    for g in range(NG):
        h_ref[g, :, :] = h[:, g * CG:(g + 1) * CG]


def _edge_mm(pa, pb, W2):
    eb = 800
    grid = (N_EDGES_C // eb,)
    return pl.pallas_call(
        _edge_mm_body,
        grid=grid,
        in_specs=[
            pl.BlockSpec((eb, D_C), lambda i: (i, 0)),
            pl.BlockSpec((eb, D_C), lambda i: (i, 0)),
            pl.BlockSpec((D_C, D_C), lambda i: (0, 0)),
        ],
        out_specs=pl.BlockSpec((NG, eb, CG), lambda i: (0, i, 0)),
        out_shape=jax.ShapeDtypeStruct((NG, N_EDGES_C, CG), jnp.float32),
    )(pa, pb, W2)


def _segmax_body(h_hbm, dst_hbm, b2_hbm, out_hbm, dstv, hbuf, b2v, acc):
    wid = lax.axis_index("s") * NC + lax.axis_index("c")
    q = wid // NG
    g = wid % NG
    lo = q * QN

    neg = jnp.full((16,), NEG_INF, dtype=jnp.float32)

    def initb(r, _):
        for half in range(CG // 16):
            acc[r, pl.ds(half * 16, 16)] = neg
        return 0

    lax.fori_loop(0, QN, initb, 0)

    def chunk(j, _):
        base = j * EC
        pltpu.sync_copy(dst_hbm.at[pl.ds(base, EC)], dstv)
        pltpu.sync_copy(h_hbm.at[g, pl.ds(base, EC)], hbuf)

        def edge16(t, _):
            dvec = dstv[pl.ds(t * 16, 16)]
            for k in range(16):
                d = dvec[k]
                e = t * 16 + k

                @pl.when((d >= lo) & (d < lo + QN))
                def _(d=d, e=e):
                    dl = d - lo
                    for half in range(CG // 16):
                        sl = pl.ds(half * 16, 16)
                        acc[dl, sl] = jnp.maximum(acc[dl, sl], hbuf[e, sl])

            return 0

        lax.fori_loop(0, EC // 16, edge16, 0)
        return 0

    lax.fori_loop(0, N_EDGES_C // EC, chunk, 0)

    pltpu.sync_copy(b2_hbm.at[pl.ds(g * CG, CG)], b2v)

    def fixrow(r, _):
        for half in range(CG // 16):
            sl = pl.ds(half * 16, 16)
            v = acc[r, sl]
            acc[r, sl] = jnp.where(v == NEG_INF, 0.0, v + b2v[sl])
        return 0

    lax.fori_loop(0, QN, fixrow, 0)
    pltpu.sync_copy(acc, out_hbm.at[wid])


@functools.partial(
    pl.kernel,
    out_type=jax.ShapeDtypeStruct((NW, QN, CG), jnp.float32),
    mesh=_sc_mesh,
    scratch_types=[
        pltpu.VMEM((EC,), jnp.int32),
        pltpu.VMEM((EC, CG), jnp.float32),
        pltpu.VMEM((CG,), jnp.float32),
        pltpu.VMEM((QN, CG), jnp.float32),
    ],
)
def _segmax(h_hbm, dst_hbm, b2_hbm, out_hbm, dstv, hbuf, b2v, acc):
    _segmax_body(h_hbm, dst_hbm, b2_hbm, out_hbm, dstv, hbuf, b2v, acc)


@jax.jit
def kernel(x, edge_index, W1, b1, W2, b2):
    src = edge_index[0]
    dst = edge_index[1]
    a, b = _node_mm(x, W1, b1)
    pa, pb = _edge_gather(a, b, dst, src)
    h = _edge_mm(pa, pb, W2)
    out4 = _segmax(h, dst, b2)
    return (
        out4.reshape(NQ, NG, QN, CG)
        .transpose(0, 2, 1, 3)
        .reshape(N_NODES_C, D_C)
    )


# speedup vs baseline: 5.1286x; 5.1286x over previous
import functools
import jax
import jax.numpy as jnp
from jax import lax
from jax.experimental import pallas as pl
from jax.experimental.pallas import tpu as pltpu
from jax.experimental.pallas import tpu_sc as plsc

N_NODES_C = 10000
N_EDGES_C = 160000
D_C = 256

NC, NS = 2, 16
NW = NC * NS
NQ = 16
NG = 2
CG = D_C // NG
QN = N_NODES_C // NQ
EPW = N_EDGES_C // NW
GC = 200
EC = 320
KD = 256
NB = 4
CAP = KD * NB
NEG_INF = float("-inf")

_sc_mesh = plsc.VectorSubcoreMesh(
    core_axis_name="c", subcore_axis_name="s", num_cores=NC, num_subcores=NS
)


def _node_mm_body(x_ref, w1_ref, b1_ref, a_ref, b_ref):
    xb = x_ref[...]
    wb = w1_ref[D_C:, :]
    wa = w1_ref[:D_C, :] - wb
    a_ref[...] = (
        jnp.dot(xb, wa, preferred_element_type=jnp.float32) + b1_ref[...]
    )
    b_ref[...] = jnp.dot(xb, wb, preferred_element_type=jnp.float32)


def _node_mm(x, W1, b1):
    nb = 400
    grid = (N_NODES_C // nb,)
    return pl.pallas_call(
        _node_mm_body,
        grid=grid,
        in_specs=[
            pl.BlockSpec((nb, D_C), lambda i: (i, 0)),
            pl.BlockSpec((2 * D_C, D_C), lambda i: (0, 0)),
            pl.BlockSpec((1, D_C), lambda i: (0, 0)),
        ],
        out_specs=[
            pl.BlockSpec((nb, D_C), lambda i: (i, 0)),
            pl.BlockSpec((nb, D_C), lambda i: (i, 0)),
        ],
        out_shape=[
            jax.ShapeDtypeStruct((N_NODES_C, D_C), jnp.float32),
            jax.ShapeDtypeStruct((N_NODES_C, D_C), jnp.float32),
        ],
    )(x, W1, b1.reshape(1, D_C))


def _gather_body(a_hbm, b_hbm, dst_hbm, src_hbm, pa_hbm, pb_hbm,
                 didx, sidx, bufa, bufb, sema, semb):
    wid = lax.axis_index("s") * NC + lax.axis_index("c")
    e0 = wid * EPW

    def chunk(j, _):
        base = e0 + j * GC
        pltpu.sync_copy(dst_hbm.at[pl.ds(base, GC)], didx)
        pltpu.sync_copy(src_hbm.at[pl.ds(base, GC)], sidx)
        cpa = pltpu.async_copy(a_hbm.at[didx], bufa, sema)
        cpb = pltpu.async_copy(b_hbm.at[sidx], bufb, semb)
        cpa.wait()
        pltpu.sync_copy(bufa, pa_hbm.at[pl.ds(base, GC)])
        cpb.wait()
        pltpu.sync_copy(bufb, pb_hbm.at[pl.ds(base, GC)])
        return 0

    lax.fori_loop(0, EPW // GC, chunk, 0)


@functools.partial(
    pl.kernel,
    out_type=[
        jax.ShapeDtypeStruct((N_EDGES_C, D_C), jnp.float32),
        jax.ShapeDtypeStruct((N_EDGES_C, D_C), jnp.float32),
    ],
    mesh=_sc_mesh,
    scratch_types=[
        pltpu.VMEM((GC,), jnp.int32),
        pltpu.VMEM((GC,), jnp.int32),
        pltpu.VMEM((GC, D_C), jnp.float32),
        pltpu.VMEM((GC, D_C), jnp.float32),
        pltpu.SemaphoreType.DMA,
        pltpu.SemaphoreType.DMA,
    ],
)
def _edge_gather(a_hbm, b_hbm, dst_hbm, src_hbm, pa_hbm, pb_hbm,
                 didx, sidx, bufa, bufb, sema, semb):
    _gather_body(a_hbm, b_hbm, dst_hbm, src_hbm, pa_hbm, pb_hbm,
                 didx, sidx, bufa, bufb, sema, semb)


def _edge_mm_body(pa_ref, pb_ref, w2_ref, h0_ref, h1_ref):
    m = jnp.maximum(pa_ref[...] + pb_ref[...], 0.0)
    h = jnp.dot(m, w2_ref[...], preferred_element_type=jnp.float32)
    h0_ref[...] = h[:, :CG]
    h1_ref[...] = h[:, CG:]


def _edge_mm(pa, pb, W2):
    eb = 800
    grid = (N_EDGES_C // eb,)
    return pl.pallas_call(
        _edge_mm_body,
        grid=grid,
        in_specs=[
            pl.BlockSpec((eb, D_C), lambda i: (i, 0)),
            pl.BlockSpec((eb, D_C), lambda i: (i, 0)),
            pl.BlockSpec((D_C, D_C), lambda i: (0, 0)),
        ],
        out_specs=[
            pl.BlockSpec((eb, CG), lambda i: (i, 0)),
            pl.BlockSpec((eb, CG), lambda i: (i, 0)),
        ],
        out_shape=[
            jax.ShapeDtypeStruct((N_EDGES_C, CG), jnp.float32),
            jax.ShapeDtypeStruct((N_EDGES_C, CG), jnp.float32),
        ],
    )(pa, pb, W2)


def _segmax_body(h0_hbm, h1_hbm, dst_hbm, b2_hbm, out_hbm,
                 dstv, eidl, dll, hbk, b2v, acc, sem):
    wid = lax.axis_index("s") * NC + lax.axis_index("c")
    q = wid // NG
    g = wid % NG
    lo = q * QN
    lanes = jnp.arange(16, dtype=jnp.int32)
    zeros = jnp.zeros((16,), dtype=jnp.int32)
    neg = jnp.full((16,), NEG_INF, dtype=jnp.float32)

    def initb(r, _):
        for half in range(CG // 16):
            acc[r, pl.ds(half * 16, 16)] = neg
        return 0

    lax.fori_loop(0, QN + 1, initb, 0)

    def initl(t, _):
        eidl[pl.ds(t * 16, 16)] = zeros
        dll[pl.ds(t * 16, 16)] = zeros + QN
        return 0

    lax.fori_loop(0, CAP // 16, initl, 0)

    def do_drain(nd):
        doff = (nd & (NB - 1)) * KD

        @pl.when(g == 0)
        def _():
            pltpu.async_copy(
                h0_hbm.at[eidl.at[pl.ds(doff, KD)]], hbk, sem
            ).wait()

        @pl.when(g != 0)
        def _():
            pltpu.async_copy(
                h1_hbm.at[eidl.at[pl.ds(doff, KD)]], hbk, sem
            ).wait()

        def hit16(tt, _):
            dv = dll[pl.ds(doff + tt * 16, 16)]
            for k in range(16):
                dl = dv[k]
                i = tt * 16 + k
                for half in range(CG // 16):
                    sl = pl.ds(half * 16, 16)
                    acc[dl, sl] = jnp.maximum(acc[dl, sl], hbk[i, sl])
            return 0

        lax.fori_loop(0, KD // 16, hit16, 0)

    def chunk(j, carry):
        cnt, nd = carry
        base = j * EC
        pltpu.sync_copy(dst_hbm.at[pl.ds(base, EC)], dstv)

        def grp(t, cnt):
            dvec = dstv[pl.ds(t * 16, 16)]
            dloc = dvec - lo
            m = dloc.astype(jnp.uint32) < jnp.uint32(QN)
            csum = plsc.cumsum(jnp.where(m, 1, 0))
            pos = (cnt + csum - 1) & (CAP - 1)
            eidv = base + t * 16 + lanes
            plsc.store_scatter(eidl, [pos], eidv, mask=m)
            plsc.store_scatter(dll, [pos], dloc, mask=m)
            return cnt + csum[15]

        cnt = lax.fori_loop(0, EC // 16, grp, cnt)
        target = cnt // KD

        @pl.when(target >= nd + 1)
        def _():
            do_drain(nd)

        @pl.when(target >= nd + 2)
        def _():
            do_drain(nd + 1)

        return cnt, target

    cnt, nd = lax.fori_loop(
        0, N_EDGES_C // EC, chunk,
        (jnp.int32(0), jnp.int32(0)),
    )
    pend = cnt - nd * KD

    @pl.when(pend > 0)
    def _():
        do_drain(nd)

    pltpu.sync_copy(b2_hbm.at[pl.ds(g * CG, CG)], b2v)

    def fixrow(r, _):
        for half in range(CG // 16):
            sl = pl.ds(half * 16, 16)
            v = acc[r, sl]
            acc[r, sl] = jnp.where(v == NEG_INF, 0.0, v + b2v[sl])
        return 0

    lax.fori_loop(0, QN, fixrow, 0)
    pltpu.sync_copy(acc.at[pl.ds(0, QN)], out_hbm.at[wid])


@functools.partial(
    pl.kernel,
    out_type=jax.ShapeDtypeStruct((NW, QN, CG), jnp.float32),
    mesh=_sc_mesh,
    compiler_params=pltpu.CompilerParams(needs_layout_passes=False),
    scratch_types=[
        pltpu.VMEM((EC,), jnp.int32),
        pltpu.VMEM((CAP,), jnp.int32),
        pltpu.VMEM((CAP,), jnp.int32),
        pltpu.VMEM((KD, CG), jnp.float32),
        pltpu.VMEM((CG,), jnp.float32),
        pltpu.VMEM((QN + 1, CG), jnp.float32),
        pltpu.SemaphoreType.DMA,
    ],
)
def _segmax(h0_hbm, h1_hbm, dst_hbm, b2_hbm, out_hbm,
            dstv, eidl, dll, hbk, b2v, acc, sem):
    _segmax_body(h0_hbm, h1_hbm, dst_hbm, b2_hbm, out_hbm,
                 dstv, eidl, dll, hbk, b2v, acc, sem)


@jax.jit
def kernel(x, edge_index, W1, b1, W2, b2):
    src = edge_index[0]
    dst = edge_index[1]
    a, b = _node_mm(x, W1, b1)
    pa, pb = _edge_gather(a, b, dst, src)
    h0, h1 = _edge_mm(pa, pb, W2)
    out4 = _segmax(h0, h1, dst, b2)
    return (
        out4.reshape(NQ, NG, QN, CG)
        .transpose(0, 2, 1, 3)
        .reshape(N_NODES_C, D_C)
    )


# speedup vs baseline: 6.2729x; 1.2231x over previous
import functools
import jax
import jax.numpy as jnp
from jax import lax
from jax.experimental import pallas as pl
from jax.experimental.pallas import tpu as pltpu
from jax.experimental.pallas import tpu_sc as plsc

N_NODES_C = 10000
N_EDGES_C = 160000
D_C = 256

NC, NS = 2, 16
NW = NC * NS
NQ = 16
NG = 2
CG = D_C // NG
QN = N_NODES_C // NQ
EPW = N_EDGES_C // NW
GC = 200
EC = 1600
NCH = N_EDGES_C // EC
KD = 256
NB = 8
CAP = KD * NB
NEG_INF = float("-inf")

_sc_mesh = plsc.VectorSubcoreMesh(
    core_axis_name="c", subcore_axis_name="s", num_cores=NC, num_subcores=NS
)


def _node_mm_body(x_ref, w1_ref, b1_ref, a_ref, b_ref):
    xb = x_ref[...]
    wb = w1_ref[D_C:, :]
    wa = w1_ref[:D_C, :] - wb
    a_ref[...] = (
        jnp.dot(xb, wa, preferred_element_type=jnp.float32) + b1_ref[...]
    )
    b_ref[...] = jnp.dot(xb, wb, preferred_element_type=jnp.float32)


def _node_mm(x, W1, b1):
    nb = 400
    grid = (N_NODES_C // nb,)
    return pl.pallas_call(
        _node_mm_body,
        grid=grid,
        in_specs=[
            pl.BlockSpec((nb, D_C), lambda i: (i, 0)),
            pl.BlockSpec((2 * D_C, D_C), lambda i: (0, 0)),
            pl.BlockSpec((1, D_C), lambda i: (0, 0)),
        ],
        out_specs=[
            pl.BlockSpec((nb, D_C), lambda i: (i, 0)),
            pl.BlockSpec((nb, D_C), lambda i: (i, 0)),
        ],
        out_shape=[
            jax.ShapeDtypeStruct((N_NODES_C, D_C), jnp.float32),
            jax.ShapeDtypeStruct((N_NODES_C, D_C), jnp.float32),
        ],
    )(x, W1, b1.reshape(1, D_C))


def _gather_body(a_hbm, b_hbm, dst_hbm, src_hbm, pa_hbm, pb_hbm,
                 didx, sidx, bufa, bufb, sema, semb):
    wid = lax.axis_index("s") * NC + lax.axis_index("c")
    e0 = wid * EPW

    def chunk(j, _):
        base = e0 + j * GC
        pltpu.sync_copy(dst_hbm.at[pl.ds(base, GC)], didx)
        pltpu.sync_copy(src_hbm.at[pl.ds(base, GC)], sidx)
        cpa = pltpu.async_copy(a_hbm.at[didx], bufa, sema)
        cpb = pltpu.async_copy(b_hbm.at[sidx], bufb, semb)
        cpa.wait()
        pltpu.sync_copy(bufa, pa_hbm.at[pl.ds(base, GC)])
        cpb.wait()
        pltpu.sync_copy(bufb, pb_hbm.at[pl.ds(base, GC)])
        return 0

    lax.fori_loop(0, EPW // GC, chunk, 0)


@functools.partial(
    pl.kernel,
    out_type=[
        jax.ShapeDtypeStruct((N_EDGES_C, D_C), jnp.float32),
        jax.ShapeDtypeStruct((N_EDGES_C, D_C), jnp.float32),
    ],
    mesh=_sc_mesh,
    scratch_types=[
        pltpu.VMEM((GC,), jnp.int32),
        pltpu.VMEM((GC,), jnp.int32),
        pltpu.VMEM((GC, D_C), jnp.float32),
        pltpu.VMEM((GC, D_C), jnp.float32),
        pltpu.SemaphoreType.DMA,
        pltpu.SemaphoreType.DMA,
    ],
)
def _edge_gather(a_hbm, b_hbm, dst_hbm, src_hbm, pa_hbm, pb_hbm,
                 didx, sidx, bufa, bufb, sema, semb):
    _gather_body(a_hbm, b_hbm, dst_hbm, src_hbm, pa_hbm, pb_hbm,
                 didx, sidx, bufa, bufb, sema, semb)


def _edge_mm_body(pa_ref, pb_ref, w2_ref, h0_ref, h1_ref):
    m = jnp.maximum(pa_ref[...] + pb_ref[...], 0.0)
    h = jnp.dot(m, w2_ref[...], preferred_element_type=jnp.float32)
    h0_ref[...] = h[:, :CG]
    h1_ref[...] = h[:, CG:]


def _edge_mm(pa, pb, W2):
    eb = 800
    grid = (N_EDGES_C // eb,)
    return pl.pallas_call(
        _edge_mm_body,
        grid=grid,
        in_specs=[
            pl.BlockSpec((eb, D_C), lambda i: (i, 0)),
            pl.BlockSpec((eb, D_C), lambda i: (i, 0)),
            pl.BlockSpec((D_C, D_C), lambda i: (0, 0)),
        ],
        out_specs=[
            pl.BlockSpec((eb, CG), lambda i: (i, 0)),
            pl.BlockSpec((eb, CG), lambda i: (i, 0)),
        ],
        out_shape=[
            jax.ShapeDtypeStruct((N_EDGES_C, CG), jnp.float32),
            jax.ShapeDtypeStruct((N_EDGES_C, CG), jnp.float32),
        ],
    )(pa, pb, W2)


def _segmax_body(h0_hbm, h1_hbm, dst_hbm, b2_hbm, out_hbm,
                 dstv, eidl, dll, hbk, b2v, acc, sem, dsem):
    wid = lax.axis_index("s") * NC + lax.axis_index("c")
    q = wid // NG
    g = wid % NG
    lo = q * QN
    lanes = jnp.arange(16, dtype=jnp.int32)
    zeros = jnp.zeros((16,), dtype=jnp.int32)
    neg = jnp.full((16,), NEG_INF, dtype=jnp.float32)

    def initb(r, _):
        for half in range(CG // 16):
            acc[r, pl.ds(half * 16, 16)] = neg
        return 0

    lax.fori_loop(0, QN + 1, initb, 0)

    def initl(t, _):
        eidl[pl.ds(t * 16, 16)] = zeros
        dll[pl.ds(t * 16, 16)] = zeros + QN
        return 0

    lax.fori_loop(0, CAP // 16, initl, 0)

    def do_drain(nd):
        doff = (nd & (NB - 1)) * KD

        @pl.when(g == 0)
        def _():
            pltpu.async_copy(
                h0_hbm.at[eidl.at[pl.ds(doff, KD)]], hbk, sem
            ).wait()

        @pl.when(g != 0)
        def _():
            pltpu.async_copy(
                h1_hbm.at[eidl.at[pl.ds(doff, KD)]], hbk, sem
            ).wait()

        def hit16(tt, _):
            dv = dll[pl.ds(doff + tt * 16, 16)]
            for k in range(16):
                dl = dv[k]
                i = tt * 16 + k
                for half in range(CG // 16):
                    sl = pl.ds(half * 16, 16)
                    acc[dl, sl] = jnp.maximum(acc[dl, sl], hbk[i, sl])
            return 0

        lax.fori_loop(0, KD // 16, hit16, 0)

    pltpu.async_copy(dst_hbm.at[pl.ds(0, EC)], dstv.at[pl.ds(0, EC)], dsem)

    def chunk(j, carry):
        cnt, nd = carry
        base = j * EC
        b = j & 1
        pltpu.make_async_copy(
            dst_hbm.at[pl.ds(base, EC)], dstv.at[pl.ds(b * EC, EC)], dsem
        ).wait()

        @pl.when(j + 1 < NCH)
        def _():
            pltpu.async_copy(
                dst_hbm.at[pl.ds(base + EC, EC)], dstv.at[pl.ds((1 - b) * EC, EC)], dsem
            )

        def grp(t, cnt):
            dvec = dstv[pl.ds(b * EC + t * 16, 16)]
            dloc = dvec - lo
            m = dloc.astype(jnp.uint32) < jnp.uint32(QN)
            csum = plsc.cumsum(jnp.where(m, 1, 0))
            pos = (cnt + csum - 1) & (CAP - 1)
            eidv = base + t * 16 + lanes
            plsc.store_scatter(eidl, [pos], eidv, mask=m)
            plsc.store_scatter(dll, [pos], dloc, mask=m)
            return cnt + csum[15]

        cnt = lax.fori_loop(0, EC // 16, grp, cnt)
        target = cnt // KD

        def dbody(i, _):
            do_drain(i)
            return 0

        lax.fori_loop(nd, target, dbody, 0)
        return cnt, target

    cnt, nd = lax.fori_loop(
        0, NCH, chunk,
        (jnp.int32(0), jnp.int32(0)),
    )
    pend = cnt - nd * KD

    @pl.when(pend > 0)
    def _():
        do_drain(nd)

    pltpu.sync_copy(b2_hbm.at[pl.ds(g * CG, CG)], b2v)

    def fixrow(r, _):
        for half in range(CG // 16):
            sl = pl.ds(half * 16, 16)
            v = acc[r, sl]
            acc[r, sl] = jnp.where(v == NEG_INF, 0.0, v + b2v[sl])
        return 0

    lax.fori_loop(0, QN, fixrow, 0)
    pltpu.sync_copy(acc.at[pl.ds(0, QN)], out_hbm.at[wid])


@functools.partial(
    pl.kernel,
    out_type=jax.ShapeDtypeStruct((NW, QN, CG), jnp.float32),
    mesh=_sc_mesh,
    compiler_params=pltpu.CompilerParams(needs_layout_passes=False),
    scratch_types=[
        pltpu.VMEM((2 * EC,), jnp.int32),
        pltpu.VMEM((CAP,), jnp.int32),
        pltpu.VMEM((CAP,), jnp.int32),
        pltpu.VMEM((KD, CG), jnp.float32),
        pltpu.VMEM((CG,), jnp.float32),
        pltpu.VMEM((QN + 1, CG), jnp.float32),
        pltpu.SemaphoreType.DMA,
        pltpu.SemaphoreType.DMA,
    ],
)
def _segmax(h0_hbm, h1_hbm, dst_hbm, b2_hbm, out_hbm,
            dstv, eidl, dll, hbk, b2v, acc, sem, dsem):
    _segmax_body(h0_hbm, h1_hbm, dst_hbm, b2_hbm, out_hbm,
                 dstv, eidl, dll, hbk, b2v, acc, sem, dsem)


@jax.jit
def kernel(x, edge_index, W1, b1, W2, b2):
    src = edge_index[0]
    dst = edge_index[1]
    a, b = _node_mm(x, W1, b1)
    pa, pb = _edge_gather(a, b, dst, src)
    h0, h1 = _edge_mm(pa, pb, W2)
    out4 = _segmax(h0, h1, dst, b2)
    return (
        out4.reshape(NQ, NG, QN, CG)
        .transpose(0, 2, 1, 3)
        .reshape(N_NODES_C, D_C)
    )


# speedup vs baseline: 6.4331x; 1.0255x over previous
import functools
import jax
import jax.numpy as jnp
from jax import lax
from jax.experimental import pallas as pl
from jax.experimental.pallas import tpu as pltpu
from jax.experimental.pallas import tpu_sc as plsc

N_NODES_C = 10000
N_EDGES_C = 160000
D_C = 256

NC, NS = 2, 16
NW = NC * NS
NQ = 16
NG = 2
CG = D_C // NG
QN = N_NODES_C // NQ
EPW = N_EDGES_C // NW
GC = 200
EC = 1600
NCH = N_EDGES_C // EC
KD = 256
NB = 8
CAP = KD * NB
NEG_INF = float("-inf")

_sc_mesh = plsc.VectorSubcoreMesh(
    core_axis_name="c", subcore_axis_name="s", num_cores=NC, num_subcores=NS
)


def _node_mm_body(x_ref, w1_ref, b1_ref, a_ref, b_ref):
    xb = x_ref[...]
    wb = w1_ref[D_C:, :]
    wa = w1_ref[:D_C, :] - wb
    a_ref[...] = (
        jnp.dot(xb, wa, preferred_element_type=jnp.float32) + b1_ref[...]
    )
    b_ref[...] = jnp.dot(xb, wb, preferred_element_type=jnp.float32)


def _node_mm(x, W1, b1):
    nb = 400
    grid = (N_NODES_C // nb,)
    return pl.pallas_call(
        _node_mm_body,
        grid=grid,
        in_specs=[
            pl.BlockSpec((nb, D_C), lambda i: (i, 0)),
            pl.BlockSpec((2 * D_C, D_C), lambda i: (0, 0)),
            pl.BlockSpec((1, D_C), lambda i: (0, 0)),
        ],
        out_specs=[
            pl.BlockSpec((nb, D_C), lambda i: (i, 0)),
            pl.BlockSpec((nb, D_C), lambda i: (i, 0)),
        ],
        out_shape=[
            jax.ShapeDtypeStruct((N_NODES_C, D_C), jnp.float32),
            jax.ShapeDtypeStruct((N_NODES_C, D_C), jnp.float32),
        ],
    )(x, W1, b1.reshape(1, D_C))


def _gather_body(a_hbm, b_hbm, dst_hbm, src_hbm, pa_hbm, pb_hbm,
                 didx, sidx, bufa, bufb, si_a, si_b, ga, gb, sa, sb):
    wid = lax.axis_index("s") * NC + lax.axis_index("c")
    e0 = wid * EPW
    nch = EPW // GC

    pltpu.async_copy(dst_hbm.at[pl.ds(e0, GC)], didx.at[pl.ds(0, GC)], si_a)
    pltpu.async_copy(src_hbm.at[pl.ds(e0, GC)], sidx.at[pl.ds(0, GC)], si_b)

    def chunk(j, _):
        base = e0 + j * GC
        b = j & 1
        dsl = didx.at[pl.ds(b * GC, GC)]
        ssl = sidx.at[pl.ds(b * GC, GC)]
        pltpu.make_async_copy(dst_hbm.at[pl.ds(base, GC)], dsl, si_a).wait()
        pltpu.make_async_copy(src_hbm.at[pl.ds(base, GC)], ssl, si_b).wait()

        @pl.when(j + 1 < nch)
        def _():
            nb = 1 - b
            pltpu.async_copy(
                dst_hbm.at[pl.ds(base + GC, GC)],
                didx.at[pl.ds(nb * GC, GC)], si_a)
            pltpu.async_copy(
                src_hbm.at[pl.ds(base + GC, GC)],
                sidx.at[pl.ds(nb * GC, GC)], si_b)

        @pl.when(j >= 1)
        def _():
            pb = base - GC
            pltpu.make_async_copy(
                bufa, pa_hbm.at[pl.ds(pb, GC)], sa).wait()

        pltpu.async_copy(a_hbm.at[dsl], bufa, ga).wait()
        pltpu.async_copy(bufa, pa_hbm.at[pl.ds(base, GC)], sa)

        @pl.when(j >= 1)
        def _():
            pb = base - GC
            pltpu.make_async_copy(
                bufb, pb_hbm.at[pl.ds(pb, GC)], sb).wait()

        pltpu.async_copy(b_hbm.at[ssl], bufb, gb).wait()
        pltpu.async_copy(bufb, pb_hbm.at[pl.ds(base, GC)], sb)
        return 0

    lax.fori_loop(0, nch, chunk, 0)
    last = e0 + (nch - 1) * GC
    pltpu.make_async_copy(bufa, pa_hbm.at[pl.ds(last, GC)], sa).wait()
    pltpu.make_async_copy(bufb, pb_hbm.at[pl.ds(last, GC)], sb).wait()


@functools.partial(
    pl.kernel,
    out_type=[
        jax.ShapeDtypeStruct((N_EDGES_C, D_C), jnp.float32),
        jax.ShapeDtypeStruct((N_EDGES_C, D_C), jnp.float32),
    ],
    mesh=_sc_mesh,
    scratch_types=[
        pltpu.VMEM((2 * GC,), jnp.int32),
        pltpu.VMEM((2 * GC,), jnp.int32),
        pltpu.VMEM((GC, D_C), jnp.float32),
        pltpu.VMEM((GC, D_C), jnp.float32),
        pltpu.SemaphoreType.DMA,
        pltpu.SemaphoreType.DMA,
        pltpu.SemaphoreType.DMA,
        pltpu.SemaphoreType.DMA,
        pltpu.SemaphoreType.DMA,
        pltpu.SemaphoreType.DMA,
    ],
)
def _edge_gather(a_hbm, b_hbm, dst_hbm, src_hbm, pa_hbm, pb_hbm,
                 didx, sidx, bufa, bufb, si_a, si_b, ga, gb, sa, sb):
    _gather_body(a_hbm, b_hbm, dst_hbm, src_hbm, pa_hbm, pb_hbm,
                 didx, sidx, bufa, bufb, si_a, si_b, ga, gb, sa, sb)


def _edge_mm_body(pa_ref, pb_ref, w2_ref, h0_ref, h1_ref):
    m = jnp.maximum(pa_ref[...] + pb_ref[...], 0.0)
    h = jnp.dot(m, w2_ref[...], preferred_element_type=jnp.float32)
    h0_ref[...] = h[:, :CG]
    h1_ref[...] = h[:, CG:]


def _edge_mm(pa, pb, W2):
    eb = 800
    grid = (N_EDGES_C // eb,)
    return pl.pallas_call(
        _edge_mm_body,
        grid=grid,
        in_specs=[
            pl.BlockSpec((eb, D_C), lambda i: (i, 0)),
            pl.BlockSpec((eb, D_C), lambda i: (i, 0)),
            pl.BlockSpec((D_C, D_C), lambda i: (0, 0)),
        ],
        out_specs=[
            pl.BlockSpec((eb, CG), lambda i: (i, 0)),
            pl.BlockSpec((eb, CG), lambda i: (i, 0)),
        ],
        out_shape=[
            jax.ShapeDtypeStruct((N_EDGES_C, CG), jnp.float32),
            jax.ShapeDtypeStruct((N_EDGES_C, CG), jnp.float32),
        ],
    )(pa, pb, W2)


def _segmax_body(h0_hbm, h1_hbm, dst_hbm, b2_hbm, out_hbm,
                 dstv, eidl, dll, hbk, b2v, acc, sem, dsem):
    wid = lax.axis_index("s") * NC + lax.axis_index("c")
    q = wid // NG
    g = wid % NG
    lo = q * QN
    lanes = jnp.arange(16, dtype=jnp.int32)
    zeros = jnp.zeros((16,), dtype=jnp.int32)
    neg = jnp.full((16,), NEG_INF, dtype=jnp.float32)

    def initb(r, _):
        for half in range(CG // 16):
            acc[r, pl.ds(half * 16, 16)] = neg
        return 0

    lax.fori_loop(0, QN + 1, initb, 0)

    def initl(t, _):
        eidl[pl.ds(t * 16, 16)] = zeros
        dll[pl.ds(t * 16, 16)] = zeros + QN
        return 0

    lax.fori_loop(0, CAP // 16, initl, 0)

    def do_drain(nd):
        doff = (nd & (NB - 1)) * KD

        @pl.when(g == 0)
        def _():
            pltpu.async_copy(
                h0_hbm.at[eidl.at[pl.ds(doff, KD)]], hbk, sem
            ).wait()

        @pl.when(g != 0)
        def _():
            pltpu.async_copy(
                h1_hbm.at[eidl.at[pl.ds(doff, KD)]], hbk, sem
            ).wait()

        def hit16(tt, _):
            dv = dll[pl.ds(doff + tt * 16, 16)]
            for k in range(16):
                dl = dv[k]
                i = tt * 16 + k
                for half in range(CG // 16):
                    sl = pl.ds(half * 16, 16)
                    acc[dl, sl] = jnp.maximum(acc[dl, sl], hbk[i, sl])
            return 0

        lax.fori_loop(0, KD // 16, hit16, 0)

    pltpu.async_copy(dst_hbm.at[pl.ds(0, EC)], dstv.at[pl.ds(0, EC)], dsem)

    def chunk(j, carry):
        cnt, nd = carry
        base = j * EC
        b = j & 1
        pltpu.make_async_copy(
            dst_hbm.at[pl.ds(base, EC)], dstv.at[pl.ds(b * EC, EC)], dsem
        ).wait()

        @pl.when(j + 1 < NCH)
        def _():
            pltpu.async_copy(
                dst_hbm.at[pl.ds(base + EC, EC)], dstv.at[pl.ds((1 - b) * EC, EC)], dsem
            )

        def grp(t, cnt):
            dvec = dstv[pl.ds(b * EC + t * 16, 16)]
            dloc = dvec - lo
            m = dloc.astype(jnp.uint32) < jnp.uint32(QN)
            csum = plsc.cumsum(jnp.where(m, 1, 0))
            pos = (cnt + csum - 1) & (CAP - 1)
            eidv = base + t * 16 + lanes
            plsc.store_scatter(eidl, [pos], eidv, mask=m)
            plsc.store_scatter(dll, [pos], dloc, mask=m)
            return cnt + csum[15]

        cnt = lax.fori_loop(0, EC // 16, grp, cnt)
        target = cnt // KD

        def dbody(i, _):
            do_drain(i)
            return 0

        lax.fori_loop(nd, target, dbody, 0)
        return cnt, target

    cnt, nd = lax.fori_loop(
        0, NCH, chunk,
        (jnp.int32(0), jnp.int32(0)),
    )
    pend = cnt - nd * KD

    @pl.when(pend > 0)
    def _():
        do_drain(nd)

    pltpu.sync_copy(b2_hbm.at[pl.ds(g * CG, CG)], b2v)

    def fixrow(r, _):
        for half in range(CG // 16):
            sl = pl.ds(half * 16, 16)
            v = acc[r, sl]
            acc[r, sl] = jnp.where(v == NEG_INF, 0.0, v + b2v[sl])
        return 0

    lax.fori_loop(0, QN, fixrow, 0)
    pltpu.sync_copy(acc.at[pl.ds(0, QN)], out_hbm.at[wid])


@functools.partial(
    pl.kernel,
    out_type=jax.ShapeDtypeStruct((NW, QN, CG), jnp.float32),
    mesh=_sc_mesh,
    compiler_params=pltpu.CompilerParams(needs_layout_passes=False),
    scratch_types=[
        pltpu.VMEM((2 * EC,), jnp.int32),
        pltpu.VMEM((CAP,), jnp.int32),
        pltpu.VMEM((CAP,), jnp.int32),
        pltpu.VMEM((KD, CG), jnp.float32),
        pltpu.VMEM((CG,), jnp.float32),
        pltpu.VMEM((QN + 1, CG), jnp.float32),
        pltpu.SemaphoreType.DMA,
        pltpu.SemaphoreType.DMA,
    ],
)
def _segmax(h0_hbm, h1_hbm, dst_hbm, b2_hbm, out_hbm,
            dstv, eidl, dll, hbk, b2v, acc, sem, dsem):
    _segmax_body(h0_hbm, h1_hbm, dst_hbm, b2_hbm, out_hbm,
                 dstv, eidl, dll, hbk, b2v, acc, sem, dsem)


@jax.jit
def kernel(x, edge_index, W1, b1, W2, b2):
    src = edge_index[0]
    dst = edge_index[1]
    a, b = _node_mm(x, W1, b1)
    pa, pb = _edge_gather(a, b, dst, src)
    h0, h1 = _edge_mm(pa, pb, W2)
    out4 = _segmax(h0, h1, dst, b2)
    return (
        out4.reshape(NQ, NG, QN, CG)
        .transpose(0, 2, 1, 3)
        .reshape(N_NODES_C, D_C)
    )
